# baseline (speedup 1.0000x reference)
import functools

import jax
import jax.numpy as jnp
from jax.experimental import pallas as pl
from jax.experimental.pallas import tpu as pltpu

_TOP_K = 8
_LB_WEIGHT = 0.01


def _body(x_ref, W1_ref, b1_ref, W2_ref, b2_ref, W3_ref, b3_ref, scale_ref,
          gates_ref, idx_ref, loss_ref, usage_acc, *, n_total, grid_n):
    i = pl.program_id(0)
    E = W3_ref.shape[1]
    BN = x_ref.shape[0]

    dot = functools.partial(
        jax.lax.dot_general,
        dimension_numbers=(((1,), (0,)), ((), ())),
        preferred_element_type=jnp.float32,
        precision=jax.lax.Precision.DEFAULT,
    )

    h = jnp.maximum(dot(x_ref[...], W1_ref[...]) + b1_ref[...], 0.0)
    h = jnp.maximum(dot(h, W2_ref[...]) + b2_ref[...], 0.0)
    logits = (dot(h, W3_ref[...]) + b3_ref[...]) * scale_ref[...]

    m = jnp.max(logits, axis=1, keepdims=True)
    e = jnp.exp(logits - m)
    p = e / jnp.sum(e, axis=1, keepdims=True)

    part = jnp.sum(p, axis=0).reshape(1, E)

    @pl.when(i == 0)
    def _init():
        usage_acc[...] = part

    @pl.when(i != 0)
    def _acc():
        usage_acc[...] += part

    col = jax.lax.broadcasted_iota(jnp.int32, (BN, E), 1)
    work = p
    vals = []
    idxs = []
    for _ in range(_TOP_K):
        mv = jnp.max(work, axis=1, keepdims=True)
        sel = jnp.min(jnp.where(work == mv, col, E), axis=1, keepdims=True)
        vals.append(mv)
        idxs.append(sel)
        work = jnp.where(col == sel, -1.0, work)
    gates = jnp.concatenate(vals, axis=1)
    gates_ref[...] = gates / jnp.sum(gates, axis=1, keepdims=True)
    idx_ref[...] = jnp.concatenate(idxs, axis=1)

    @pl.when(i == grid_n - 1)
    def _loss():
        usage = usage_acc[...] / jnp.float32(n_total)
        u = jnp.float32(1.0 / E)
        kl = jnp.sum(u * (jnp.log(u) - jnp.log(usage + 1e-8))) / E
        loss_ref[0, 0] = kl * _LB_WEIGHT


def kernel(x, W1, b1, W2, b2, W3, b3, expert_importance, log_temperature):
    N, D = x.shape
    H = W1.shape[1]
    E = W3.shape[1]
    BN = min(512, N)
    grid_n = N // BN

    scale = (expert_importance * jnp.exp(-log_temperature)).reshape(1, E)

    gates, idx, loss = pl.pallas_call(
        functools.partial(_body, n_total=N, grid_n=grid_n),
        grid=(grid_n,),
        in_specs=[
            pl.BlockSpec((BN, D), lambda i: (i, 0)),
            pl.BlockSpec((D, H), lambda i: (0, 0)),
            pl.BlockSpec((1, H), lambda i: (0, 0)),
            pl.BlockSpec((H, H), lambda i: (0, 0)),
            pl.BlockSpec((1, H), lambda i: (0, 0)),
            pl.BlockSpec((H, E), lambda i: (0, 0)),
            pl.BlockSpec((1, E), lambda i: (0, 0)),
            pl.BlockSpec((1, E), lambda i: (0, 0)),
        ],
        out_specs=[
            pl.BlockSpec((BN, _TOP_K), lambda i: (i, 0)),
            pl.BlockSpec((BN, _TOP_K), lambda i: (i, 0)),
            pl.BlockSpec(memory_space=pltpu.SMEM),
        ],
        out_shape=[
            jax.ShapeDtypeStruct((N, _TOP_K), jnp.float32),
            jax.ShapeDtypeStruct((N, _TOP_K), jnp.int32),
            jax.ShapeDtypeStruct((1, 1), jnp.float32),
        ],
        scratch_shapes=[pltpu.VMEM((1, E), jnp.float32)],
    )(x, W1, b1.reshape(1, H), W2, b2.reshape(1, H), W3, b3.reshape(1, E),
      scale)

    return gates, idx, loss.reshape(())


# speedup vs baseline: 1.4679x; 1.4679x over previous
import functools

import jax
import jax.numpy as jnp
from jax.experimental import pallas as pl
from jax.experimental.pallas import tpu as pltpu

_TOP_K = 8
_LB_WEIGHT = 0.01


def _body(x_ref, W1_ref, b1_ref, W2_ref, b2_ref, W3_ref, b3_ref, scale_ref,
          gates_ref, idx_ref, loss_ref, usage_acc, *, n_total, grid_n):
    i = pl.program_id(0)
    E = W3_ref.shape[1]
    BN = x_ref.shape[0]

    dot = functools.partial(
        jax.lax.dot_general,
        dimension_numbers=(((1,), (0,)), ((), ())),
        preferred_element_type=jnp.float32,
        precision=jax.lax.Precision.DEFAULT,
    )

    h = jnp.maximum(dot(x_ref[...], W1_ref[...]) + b1_ref[...], 0.0)
    h = jnp.maximum(dot(h, W2_ref[...]) + b2_ref[...], 0.0)
    logits = (dot(h, W3_ref[...]) + b3_ref[...]) * scale_ref[...]

    lt = logits.T

    m = jnp.max(lt, axis=0, keepdims=True)
    e = jnp.exp(lt - m)
    p = e / jnp.sum(e, axis=0, keepdims=True)

    part = jnp.sum(p, axis=1).reshape(1, E)

    @pl.when(i == 0)
    def _init():
        usage_acc[...] = part

    @pl.when(i != 0)
    def _acc():
        usage_acc[...] += part

    row = jax.lax.broadcasted_iota(jnp.int32, (E, BN), 0)
    work = p
    vals = []
    idxs = []
    for _ in range(_TOP_K):
        mv = jnp.max(work, axis=0, keepdims=True)
        sel = jnp.min(jnp.where(work == mv, row, E), axis=0, keepdims=True)
        vals.append(mv)
        idxs.append(sel)
        work = jnp.where(row == sel, -1.0, work)
    gates = jnp.concatenate(vals, axis=0)
    gates_ref[...] = gates / jnp.sum(gates, axis=0, keepdims=True)
    idx_ref[...] = jnp.concatenate(idxs, axis=0)

    @pl.when(i == grid_n - 1)
    def _loss():
        usage = usage_acc[...] / jnp.float32(n_total)
        u = jnp.float32(1.0 / E)
        kl = jnp.sum(u * (jnp.log(u) - jnp.log(usage + 1e-8))) / E
        loss_ref[0, 0] = kl * _LB_WEIGHT


def kernel(x, W1, b1, W2, b2, W3, b3, expert_importance, log_temperature):
    N, D = x.shape
    H = W1.shape[1]
    E = W3.shape[1]
    BN = min(512, N)
    grid_n = N // BN

    scale = (expert_importance * jnp.exp(-log_temperature)).reshape(1, E)

    gates_t, idx_t, loss = pl.pallas_call(
        functools.partial(_body, n_total=N, grid_n=grid_n),
        grid=(grid_n,),
        in_specs=[
            pl.BlockSpec((BN, D), lambda i: (i, 0)),
            pl.BlockSpec((D, H), lambda i: (0, 0)),
            pl.BlockSpec((1, H), lambda i: (0, 0)),
            pl.BlockSpec((H, H), lambda i: (0, 0)),
            pl.BlockSpec((1, H), lambda i: (0, 0)),
            pl.BlockSpec((H, E), lambda i: (0, 0)),
            pl.BlockSpec((1, E), lambda i: (0, 0)),
            pl.BlockSpec((1, E), lambda i: (0, 0)),
        ],
        out_specs=[
            pl.BlockSpec((_TOP_K, BN), lambda i: (0, i)),
            pl.BlockSpec((_TOP_K, BN), lambda i: (0, i)),
            pl.BlockSpec(memory_space=pltpu.SMEM),
        ],
        out_shape=[
            jax.ShapeDtypeStruct((_TOP_K, N), jnp.float32),
            jax.ShapeDtypeStruct((_TOP_K, N), jnp.int32),
            jax.ShapeDtypeStruct((1, 1), jnp.float32),
        ],
        scratch_shapes=[pltpu.VMEM((1, E), jnp.float32)],
    )(x, W1, b1.reshape(1, H), W2, b2.reshape(1, H), W3, b3.reshape(1, E),
      scale)

    return gates_t.T, idx_t.T, loss.reshape(())


# speedup vs baseline: 1.5978x; 1.0884x over previous
import functools

import jax
import jax.numpy as jnp
from jax.experimental import pallas as pl
from jax.experimental.pallas import tpu as pltpu

_TOP_K = 8
_LB_WEIGHT = 0.01


def _body(x_ref, W1_ref, b1_ref, W2_ref, b2_ref, W3_ref, b3_ref, scale_ref,
          gates_ref, idx_ref, loss_ref, usage_acc, *, n_total, grid_n):
    i = pl.program_id(0)
    E = W3_ref.shape[1]
    BN = x_ref.shape[0]

    dot = functools.partial(
        jax.lax.dot_general,
        dimension_numbers=(((1,), (0,)), ((), ())),
        preferred_element_type=jnp.float32,
        precision=jax.lax.Precision.DEFAULT,
    )

    h = jnp.maximum(dot(x_ref[...], W1_ref[...]) + b1_ref[...], 0.0)
    h = jnp.maximum(dot(h, W2_ref[...]) + b2_ref[...], 0.0)
    logits = (dot(h, W3_ref[...]) + b3_ref[...]) * scale_ref[...]

    lt = logits.T

    m = jnp.max(lt, axis=0, keepdims=True)
    e = jnp.exp(lt - m)
    p = e / jnp.sum(e, axis=0, keepdims=True)

    part = jnp.sum(p, axis=1).reshape(1, E)

    @pl.when(i == 0)
    def _init():
        usage_acc[...] = part

    @pl.when(i != 0)
    def _acc():
        usage_acc[...] += part

    row = jax.lax.broadcasted_iota(jnp.int32, (E, BN), 0)
    work = p
    vals = []
    idxs = []
    for _ in range(_TOP_K):
        mv = jnp.max(work, axis=0, keepdims=True)
        sel = jnp.min(jnp.where(work == mv, row, E), axis=0, keepdims=True)
        vals.append(mv)
        idxs.append(sel)
        work = jnp.where(row == sel, -1.0, work)
    gates = jnp.concatenate(vals, axis=0)
    gates_ref[...] = gates / jnp.sum(gates, axis=0, keepdims=True)
    idx_ref[...] = jnp.concatenate(idxs, axis=0)

    @pl.when(i == grid_n - 1)
    def _loss():
        usage = usage_acc[...] / jnp.float32(n_total)
        u = jnp.float32(1.0 / E)
        kl = jnp.sum(u * (jnp.log(u) - jnp.log(usage + 1e-8))) / E
        loss_ref[0, 0] = kl * _LB_WEIGHT


def kernel(x, W1, b1, W2, b2, W3, b3, expert_importance, log_temperature):
    N, D = x.shape
    H = W1.shape[1]
    E = W3.shape[1]
    BN = min(1024, N)
    grid_n = N // BN

    scale = (expert_importance * jnp.exp(-log_temperature)).reshape(1, E)

    gates_t, idx_t, loss = pl.pallas_call(
        functools.partial(_body, n_total=N, grid_n=grid_n),
        grid=(grid_n,),
        in_specs=[
            pl.BlockSpec((BN, D), lambda i: (i, 0)),
            pl.BlockSpec((D, H), lambda i: (0, 0)),
            pl.BlockSpec((1, H), lambda i: (0, 0)),
            pl.BlockSpec((H, H), lambda i: (0, 0)),
            pl.BlockSpec((1, H), lambda i: (0, 0)),
            pl.BlockSpec((H, E), lambda i: (0, 0)),
            pl.BlockSpec((1, E), lambda i: (0, 0)),
            pl.BlockSpec((1, E), lambda i: (0, 0)),
        ],
        out_specs=[
            pl.BlockSpec((_TOP_K, BN), lambda i: (0, i)),
            pl.BlockSpec((_TOP_K, BN), lambda i: (0, i)),
            pl.BlockSpec(memory_space=pltpu.SMEM),
        ],
        out_shape=[
            jax.ShapeDtypeStruct((_TOP_K, N), jnp.float32),
            jax.ShapeDtypeStruct((_TOP_K, N), jnp.int32),
            jax.ShapeDtypeStruct((1, 1), jnp.float32),
        ],
        scratch_shapes=[pltpu.VMEM((1, E), jnp.float32)],
    )(x, W1, b1.reshape(1, H), W2, b2.reshape(1, H), W3, b3.reshape(1, E),
      scale)

    return gates_t.T, idx_t.T, loss.reshape(())
